# baseline (speedup 1.0000x reference)
import jax
import jax.numpy as jnp
from jax.experimental import pallas as pl
from jax.experimental.pallas import tpu as pltpu

EPS = 1e-5


def _make_fused_kernel(n, h, w, ho, wo, L, m_valid):
    def _body(x_ref, w_ref, g_ref, b_ref, o_ref, y_scr, st_scr):
        i = pl.program_id(0)
        cin = x_ref.shape[1]
        cout = o_ref.shape[1]

        @pl.when(i < n)
        def _conv():
            xf = x_ref[0].astype(jnp.bfloat16).reshape(cin, h * w)
            acc = jnp.zeros((cout, L), jnp.float32)
            for kh in range(3):
                for kw in range(3):
                    off = kh * w + kw
                    acc2 = jnp.dot(w_ref[kh * 3 + kw], xf[:, off:off + L],
                                   preferred_element_type=jnp.float32)
                    acc = acc + acc2
            y_scr[pl.ds(i, 1)] = acc.astype(jnp.bfloat16)[None]

            col = jax.lax.broadcasted_iota(jnp.int32, (1, L), 1)
            mask = (col % w) < wo
            accm = jnp.where(mask, acc, 0.0)
            s = jnp.sum(accm, axis=1, keepdims=True)
            q = jnp.sum(accm * acc, axis=1, keepdims=True)
            sq = jnp.concatenate([s, q], axis=1)
            prev = jnp.where(i == 0, 0.0, st_scr[...])
            st_scr[...] = prev + sq

        @pl.when(i >= n)
        def _bn():
            tot = st_scr[:, 0:1]
            tsq = st_scr[:, 1:2]
            mean = tot / m_valid
            var = jnp.maximum(tsq / m_valid - mean * mean, 0.0)
            inv = jax.lax.rsqrt(var + EPS)
            scale = g_ref[...] * inv
            shift = b_ref[...] - mean * scale

            yb = y_scr[pl.ds(i - n, 1)][0]
            pad = ho * w - L
            ybp = jnp.concatenate(
                [yb, jnp.zeros((cout, pad), jnp.bfloat16)], axis=1)
            z3 = ybp.reshape(cout, ho, w)[:, :, :wo].astype(jnp.float32)
            s3 = scale.reshape(cout, 1, 1)
            t3 = shift.reshape(cout, 1, 1)
            o_ref[0] = jnp.maximum(z3 * s3 + t3, 0.0)

    return _body


def kernel(x_nchw, w_oihw, bias, gamma, beta):
    del bias
    n, cin, h, w = x_nchw.shape
    cout = w_oihw.shape[0]
    ho, wo = h - 2, w - 2
    L = ho * w - (w - wo)

    w_taps = jnp.transpose(w_oihw, (2, 3, 0, 1)).reshape(9, cout, cin)
    w_taps = w_taps.astype(jnp.bfloat16)
    g_col = gamma.reshape(cout, 1)
    b_col = beta.reshape(cout, 1)

    out = pl.pallas_call(
        _make_fused_kernel(n, h, w, ho, wo, L, float(n * ho * wo)),
        out_shape=jax.ShapeDtypeStruct((n, cout, ho, wo), jnp.float32),
        grid=(2 * n,),
        in_specs=[
            pl.BlockSpec((1, cin, h, w),
                         lambda i: (jnp.minimum(i, n - 1), 0, 0, 0)),
            pl.BlockSpec((9, cout, cin), lambda i: (0, 0, 0)),
            pl.BlockSpec((cout, 1), lambda i: (0, 0)),
            pl.BlockSpec((cout, 1), lambda i: (0, 0)),
        ],
        out_specs=pl.BlockSpec((1, cout, ho, wo),
                               lambda i: (jnp.maximum(i - n, 0), 0, 0, 0)),
        scratch_shapes=[
            pltpu.VMEM((n, cout, L), jnp.bfloat16),
            pltpu.VMEM((cout, 2), jnp.float32),
        ],
        compiler_params=pltpu.CompilerParams(
            dimension_semantics=("arbitrary",)),
    )(x_nchw, w_taps, g_col, b_col)

    return out


# speedup vs baseline: 1.1468x; 1.1468x over previous
import jax
import jax.numpy as jnp
from jax.experimental import pallas as pl
from jax.experimental.pallas import tpu as pltpu

EPS = 1e-5


def _make_fused_kernel(n, h, w, ho, wo, L, m_valid):
    def _body(x_ref, w_ref, g_ref, b_ref, o_ref, y_scr, st_scr):
        i = pl.program_id(0)
        cout = o_ref.shape[3]

        @pl.when(i < n)
        def _conv():
            xb = x_ref[0].astype(jnp.bfloat16)
            acc = jnp.zeros((L, cout), jnp.float32)
            for kh in range(3):
                for kw in range(3):
                    off = kh * w + kw
                    acc = acc + jnp.dot(
                        xb[off:off + L, :], w_ref[kh * 3 + kw],
                        preferred_element_type=jnp.float32)

            y_scr[pl.ds(i, 1), :L] = acc.astype(jnp.bfloat16)[None]
            y_scr[pl.ds(i, 1), L:] = jnp.zeros(
                (1, ho * w - L, cout), jnp.bfloat16)

            row = jax.lax.broadcasted_iota(jnp.int32, (L, 1), 0)
            mask = (row % w) < wo
            accm = jnp.where(mask, acc, 0.0)
            s = jnp.sum(accm, axis=0, keepdims=True)
            q = jnp.sum(accm * acc, axis=0, keepdims=True)
            sq = jnp.concatenate([s, q], axis=0)
            prev = jnp.where(i == 0, 0.0, st_scr[0:2])
            st_scr[0:2] = prev + sq

        @pl.when(i >= n)
        def _bn():
            tot = st_scr[0:1]
            tsq = st_scr[1:2]
            mean = tot / m_valid
            var = jnp.maximum(tsq / m_valid - mean * mean, 0.0)
            inv = jax.lax.rsqrt(var + EPS)
            scale = g_ref[...] * inv
            shift = b_ref[...] - mean * scale

            yb = y_scr[pl.ds(i - n, 1)][0]
            z = yb.astype(jnp.float32) * scale + shift
            z = jnp.maximum(z, 0.0)
            o_ref[0] = z.reshape(ho, w, cout)

    return _body


def kernel(x_nchw, w_oihw, bias, gamma, beta):
    del bias
    n, cin, h, w = x_nchw.shape
    cout = w_oihw.shape[0]
    ho, wo = h - 2, w - 2
    L = ho * w - (w - wo)

    x_flat = jnp.transpose(x_nchw, (0, 2, 3, 1)).reshape(n, h * w, cin)

    w_taps = jnp.transpose(w_oihw, (2, 3, 1, 0)).reshape(9, cin, cout)
    w_taps = w_taps.astype(jnp.bfloat16)
    g_row = gamma.reshape(1, cout)
    b_row = beta.reshape(1, cout)

    out_p = pl.pallas_call(
        _make_fused_kernel(n, h, w, ho, wo, L, float(n * ho * wo)),
        out_shape=jax.ShapeDtypeStruct((n, ho, w, cout), jnp.float32),
        grid=(2 * n,),
        in_specs=[
            pl.BlockSpec((1, h * w, cin),
                         lambda i: (jnp.minimum(i, n - 1), 0, 0)),
            pl.BlockSpec((9, cin, cout), lambda i: (0, 0, 0)),
            pl.BlockSpec((1, cout), lambda i: (0, 0)),
            pl.BlockSpec((1, cout), lambda i: (0, 0)),
        ],
        out_specs=pl.BlockSpec((1, ho, w, cout),
                               lambda i: (jnp.maximum(i - n, 0), 0, 0, 0)),
        scratch_shapes=[
            pltpu.VMEM((n, ho * w, cout), jnp.bfloat16),
            pltpu.VMEM((8, cout), jnp.float32),
        ],
        compiler_params=pltpu.CompilerParams(
            dimension_semantics=("arbitrary",)),
    )(x_flat, w_taps, g_row, b_row)

    return jnp.transpose(out_p[:, :, :wo, :], (0, 3, 1, 2))


# speedup vs baseline: 2.1745x; 1.8962x over previous
import jax
import jax.numpy as jnp
from jax.experimental import pallas as pl
from jax.experimental.pallas import tpu as pltpu

EPS = 1e-5


def _make_fused_kernel(n, h, w, ho, wo, L, m_valid):
    def _body(x_ref, w_ref, g_ref, b_ref, o_ref, y_scr, st_scr):
        i = pl.program_id(0)
        cout = o_ref.shape[3]

        @pl.when(i < n)
        def _conv():
            xs = x_ref[0]
            acc = jnp.zeros((L, cout), jnp.float32)
            for kh in range(3):
                for kw in range(3):
                    off = kh * w + kw
                    acc = acc + jnp.dot(
                        xs[off:off + L, :], w_ref[kh * 3 + kw],
                        preferred_element_type=jnp.float32)

            accp = jnp.concatenate(
                [acc, jnp.zeros((ho * w - L, cout), jnp.float32)], axis=0)
            y_scr[pl.ds(i, 1)] = accp.astype(jnp.bfloat16).reshape(
                1, ho, w, cout)

            row = jax.lax.broadcasted_iota(jnp.int32, (L, 1), 0)
            mask = (row % w) < wo
            accm = jnp.where(mask, acc, 0.0)
            s = jnp.sum(accm, axis=0, keepdims=True)
            q = jnp.sum(accm * acc, axis=0, keepdims=True)
            sq = jnp.concatenate([s, q], axis=0)
            prev = jnp.where(i == 0, 0.0, st_scr[0:2])
            st_scr[0:2] = prev + sq

        @pl.when(i >= n)
        def _bn_row():
            j = i - n
            tot = st_scr[0:1]
            tsq = st_scr[1:2]
            mean = tot / m_valid
            var = jnp.maximum(tsq / m_valid - mean * mean, 0.0)
            inv = jax.lax.rsqrt(var + EPS)
            scale = (g_ref[...] * inv).reshape(1, 1, cout)
            shift = (b_ref[...] - mean * g_ref[...] * inv).reshape(1, 1, cout)

            slab = y_scr[:, j, :wo, :]
            z = jnp.maximum(slab.astype(jnp.float32) * scale + shift, 0.0)
            o_ref[0] = jnp.swapaxes(z, 0, 1)

    return _body


def kernel(x_nchw, w_oihw, bias, gamma, beta):
    del bias
    n, cin, h, w = x_nchw.shape
    cout = w_oihw.shape[0]
    ho, wo = h - 2, w - 2
    L = ho * w - (w - wo)

    x_flat = jnp.transpose(x_nchw, (0, 2, 3, 1)).reshape(n, h * w, cin)

    w_taps = jnp.transpose(w_oihw, (2, 3, 1, 0)).reshape(9, cin, cout)
    g_row = gamma.reshape(1, cout)
    b_row = beta.reshape(1, cout)

    out_p = pl.pallas_call(
        _make_fused_kernel(n, h, w, ho, wo, L, float(n * ho * wo)),
        out_shape=jax.ShapeDtypeStruct((ho, wo, n, cout), jnp.float32),
        grid=(n + ho,),
        in_specs=[
            pl.BlockSpec((1, h * w, cin),
                         lambda i: (jnp.minimum(i, n - 1), 0, 0)),
            pl.BlockSpec((9, cin, cout), lambda i: (0, 0, 0)),
            pl.BlockSpec((1, cout), lambda i: (0, 0)),
            pl.BlockSpec((1, cout), lambda i: (0, 0)),
        ],
        out_specs=pl.BlockSpec((1, wo, n, cout),
                               lambda i: (jnp.maximum(i - n, 0), 0, 0, 0)),
        scratch_shapes=[
            pltpu.VMEM((n, ho, w, cout), jnp.bfloat16),
            pltpu.VMEM((8, cout), jnp.float32),
        ],
        compiler_params=pltpu.CompilerParams(
            dimension_semantics=("arbitrary",)),
    )(x_flat, w_taps, g_row, b_row)

    return jnp.transpose(out_p, (2, 3, 0, 1))


# speedup vs baseline: 2.7487x; 1.2640x over previous
import jax
import jax.numpy as jnp
from jax.experimental import pallas as pl
from jax.experimental.pallas import tpu as pltpu

EPS = 1e-5


def _make_fused_kernel(n, h, w, ho, wo, L, m_valid, RC):
    def _body(x_ref, w_ref, g_ref, b_ref, o_ref, y_scr, st_scr):
        i = pl.program_id(0)
        cout = o_ref.shape[3]

        @pl.when(i < n)
        def _conv():
            xs = x_ref[0]
            acc = jnp.zeros((L, cout), jnp.float32)
            for kh in range(3):
                for kw in range(3):
                    off = kh * w + kw
                    acc = acc + jnp.dot(
                        xs[off:off + L, :], w_ref[kh * 3 + kw],
                        preferred_element_type=jnp.float32)

            accp = jnp.concatenate(
                [acc, jnp.zeros((ho * w - L, cout), jnp.float32)], axis=0)
            y_scr[pl.ds(i, 1)] = accp.astype(jnp.bfloat16).reshape(
                1, ho, w, cout)

            row = jax.lax.broadcasted_iota(jnp.int32, (L, 1), 0)
            mask = (row % w) < wo
            accm = jnp.where(mask, acc, 0.0)
            s = jnp.sum(accm, axis=0, keepdims=True)
            q = jnp.sum(accm * acc, axis=0, keepdims=True)
            sq = jnp.concatenate([s, q], axis=0)
            prev = jnp.where(i == 0, 0.0, st_scr[0:2])
            st_scr[0:2] = prev + sq

        @pl.when(i >= n)
        def _bn_rows():
            j = i - n
            tot = st_scr[0:1]
            tsq = st_scr[1:2]
            mean = tot / m_valid
            var = jnp.maximum(tsq / m_valid - mean * mean, 0.0)
            inv = jax.lax.rsqrt(var + EPS)
            scale = (g_ref[...] * inv).reshape(1, 1, 1, cout)
            shift = (b_ref[...] - mean * g_ref[...] * inv).reshape(
                1, 1, 1, cout)

            slab = y_scr[:, pl.ds(j * RC, RC), :wo, :]
            z = jnp.maximum(slab.astype(jnp.float32) * scale + shift, 0.0)
            o_ref[...] = jnp.transpose(z, (1, 2, 0, 3))

    return _body


def kernel(x_nchw, w_oihw, bias, gamma, beta):
    del bias
    n, cin, h, w = x_nchw.shape
    cout = w_oihw.shape[0]
    ho, wo = h - 2, w - 2
    L = ho * w - (w - wo)
    RC = next(r for r in (9, 6, 3, 2, 1) if ho % r == 0)

    x_flat = jnp.transpose(x_nchw, (0, 2, 3, 1)).reshape(n, h * w, cin)

    w_taps = jnp.transpose(w_oihw, (2, 3, 1, 0)).reshape(9, cin, cout)
    g_row = gamma.reshape(1, cout)
    b_row = beta.reshape(1, cout)

    out_p = pl.pallas_call(
        _make_fused_kernel(n, h, w, ho, wo, L, float(n * ho * wo), RC),
        out_shape=jax.ShapeDtypeStruct((ho, wo, n, cout), jnp.float32),
        grid=(n + ho // RC,),
        in_specs=[
            pl.BlockSpec((1, h * w, cin),
                         lambda i: (jnp.minimum(i, n - 1), 0, 0)),
            pl.BlockSpec((9, cin, cout), lambda i: (0, 0, 0)),
            pl.BlockSpec((1, cout), lambda i: (0, 0)),
            pl.BlockSpec((1, cout), lambda i: (0, 0)),
        ],
        out_specs=pl.BlockSpec((RC, wo, n, cout),
                               lambda i, _rc=RC: (jnp.maximum(i - n, 0), 0, 0, 0)),
        scratch_shapes=[
            pltpu.VMEM((n, ho, w, cout), jnp.bfloat16),
            pltpu.VMEM((8, cout), jnp.float32),
        ],
        compiler_params=pltpu.CompilerParams(
            dimension_semantics=("arbitrary",)),
    )(x_flat, w_taps, g_row, b_row)

    return jnp.transpose(out_p, (2, 3, 0, 1))


# speedup vs baseline: 2.9568x; 1.0757x over previous
import jax
import jax.numpy as jnp
from jax.experimental import pallas as pl
from jax.experimental.pallas import tpu as pltpu

EPS = 1e-5


def _make_fused_kernel(n, h, w, ho, wo, L, m_valid, RC, IC):
    def _body(x_ref, w_ref, g_ref, b_ref, o_ref, y_scr, st_scr):
        i = pl.program_id(0)
        cout = o_ref.shape[3]

        @pl.when(i < n // IC)
        def _conv():
            ssum = jnp.zeros((1, cout), jnp.float32)
            sssq = jnp.zeros((1, cout), jnp.float32)
            for k in range(IC):
                xs = x_ref[k]
                acc = jnp.zeros((L, cout), jnp.float32)
                for kh in range(3):
                    for kw in range(3):
                        off = kh * w + kw
                        acc = acc + jnp.dot(
                            xs[off:off + L, :], w_ref[kh * 3 + kw],
                            preferred_element_type=jnp.float32)

                accp = jnp.concatenate(
                    [acc, jnp.zeros((ho * w - L, cout), jnp.float32)], axis=0)
                y_scr[pl.ds(i * IC + k, 1)] = accp.astype(
                    jnp.bfloat16).reshape(1, ho, w, cout)

                row = jax.lax.broadcasted_iota(jnp.int32, (L, 1), 0)
                mask = (row % w) < wo
                accm = jnp.where(mask, acc, 0.0)
                ssum = ssum + jnp.sum(accm, axis=0, keepdims=True)
                sssq = sssq + jnp.sum(accm * acc, axis=0, keepdims=True)
            sq = jnp.concatenate([ssum, sssq], axis=0)
            prev = jnp.where(i == 0, 0.0, st_scr[0:2])
            st_scr[0:2] = prev + sq

        @pl.when(i >= n // IC)
        def _bn_rows():
            j = i - n // IC
            tot = st_scr[0:1]
            tsq = st_scr[1:2]
            mean = tot / m_valid
            var = jnp.maximum(tsq / m_valid - mean * mean, 0.0)
            inv = jax.lax.rsqrt(var + EPS)
            scale = (g_ref[...] * inv).reshape(1, 1, 1, cout)
            shift = (b_ref[...] - mean * g_ref[...] * inv).reshape(
                1, 1, 1, cout)

            slab = y_scr[:, pl.ds(j * RC, RC), :wo, :]
            z = jnp.maximum(slab.astype(jnp.float32) * scale + shift, 0.0)
            o_ref[...] = jnp.transpose(z, (1, 2, 0, 3))

    return _body


def kernel(x_nchw, w_oihw, bias, gamma, beta):
    del bias
    n, cin, h, w = x_nchw.shape
    cout = w_oihw.shape[0]
    ho, wo = h - 2, w - 2
    L = ho * w - (w - wo)
    RC = next(r for r in (9, 6, 3, 2, 1) if ho % r == 0)
    IC = 2 if n % 2 == 0 else 1

    x_flat = jnp.transpose(x_nchw, (0, 2, 3, 1)).reshape(n, h * w, cin)

    w_taps = jnp.transpose(w_oihw, (2, 3, 1, 0)).reshape(9, cin, cout)
    g_row = gamma.reshape(1, cout)
    b_row = beta.reshape(1, cout)

    out_p = pl.pallas_call(
        _make_fused_kernel(n, h, w, ho, wo, L, float(n * ho * wo), RC, IC),
        out_shape=jax.ShapeDtypeStruct((ho, wo, n, cout), jnp.float32),
        grid=(n // IC + ho // RC,),
        in_specs=[
            pl.BlockSpec((IC, h * w, cin),
                         lambda i, _ic=IC: (jnp.minimum(i, n // _ic - 1), 0, 0)),
            pl.BlockSpec((9, cin, cout), lambda i: (0, 0, 0)),
            pl.BlockSpec((1, cout), lambda i: (0, 0)),
            pl.BlockSpec((1, cout), lambda i: (0, 0)),
        ],
        out_specs=pl.BlockSpec((RC, wo, n, cout),
                               lambda i, _ic=IC: (jnp.maximum(i - n // _ic, 0),
                                                  0, 0, 0)),
        scratch_shapes=[
            pltpu.VMEM((n, ho, w, cout), jnp.bfloat16),
            pltpu.VMEM((8, cout), jnp.float32),
        ],
        compiler_params=pltpu.CompilerParams(
            dimension_semantics=("arbitrary",)),
    )(x_flat, w_taps, g_row, b_row)

    return jnp.transpose(out_p, (2, 3, 0, 1))


# speedup vs baseline: 2.9988x; 1.0142x over previous
import jax
import jax.numpy as jnp
from jax.experimental import pallas as pl
from jax.experimental.pallas import tpu as pltpu

EPS = 1e-5


def _make_fused_kernel(n, h, w, ho, wo, L, m_valid, RC, IC):
    def _body(x_ref, w_ref, g_ref, b_ref, o_ref, y_scr, st_scr):
        i = pl.program_id(0)
        cout = o_ref.shape[3]

        @pl.when(i < n // IC)
        def _conv():
            ssum = jnp.zeros((1, cout), jnp.float32)
            sssq = jnp.zeros((1, cout), jnp.float32)
            for k in range(IC):
                xs = x_ref[k]
                acc = jnp.zeros((L, cout), jnp.float32)
                for kh in range(3):
                    for kw in range(3):
                        off = kh * w + kw
                        acc = acc + jnp.dot(
                            xs[off:off + L, :], w_ref[kh * 3 + kw],
                            preferred_element_type=jnp.float32)

                accp = jnp.concatenate(
                    [acc, jnp.zeros((ho * w - L, cout), jnp.float32)], axis=0)
                y_scr[pl.ds(i * IC + k, 1)] = accp.astype(
                    jnp.bfloat16).reshape(1, ho, w, cout)

                row = jax.lax.broadcasted_iota(jnp.int32, (L, 1), 0)
                mask = (row % w) < wo
                accm = jnp.where(mask, acc, 0.0)
                ssum = ssum + jnp.sum(accm, axis=0, keepdims=True)
                sssq = sssq + jnp.sum(accm * acc, axis=0, keepdims=True)
            sq = jnp.concatenate([ssum, sssq], axis=0)
            prev = jnp.where(i == 0, 0.0, st_scr[0:2])
            st_scr[0:2] = prev + sq

        @pl.when(i >= n // IC)
        def _bn_rows():
            j = i - n // IC
            tot = st_scr[0:1]
            tsq = st_scr[1:2]
            mean = tot / m_valid
            var = jnp.maximum(tsq / m_valid - mean * mean, 0.0)
            inv = jax.lax.rsqrt(var + EPS)
            scale = (g_ref[...] * inv).reshape(1, 1, 1, cout)
            shift = (b_ref[...] - mean * g_ref[...] * inv).reshape(
                1, 1, 1, cout)

            slab = y_scr[:, pl.ds(j * RC, RC), :wo, :]
            z = jnp.maximum(slab.astype(jnp.float32) * scale + shift, 0.0)
            o_ref[...] = jnp.transpose(z, (1, 2, 0, 3))

    return _body


def kernel(x_nchw, w_oihw, bias, gamma, beta):
    del bias
    n, cin, h, w = x_nchw.shape
    cout = w_oihw.shape[0]
    ho, wo = h - 2, w - 2
    L = ho * w - (w - wo)
    RC = next(r for r in (9, 6, 3, 2, 1) if ho % r == 0)
    IC = next(c for c in (4, 2, 1) if n % c == 0)

    x_flat = jnp.transpose(x_nchw, (0, 2, 3, 1)).reshape(n, h * w, cin)

    w_taps = jnp.transpose(w_oihw, (2, 3, 1, 0)).reshape(9, cin, cout)
    g_row = gamma.reshape(1, cout)
    b_row = beta.reshape(1, cout)

    out_p = pl.pallas_call(
        _make_fused_kernel(n, h, w, ho, wo, L, float(n * ho * wo), RC, IC),
        out_shape=jax.ShapeDtypeStruct((ho, wo, n, cout), jnp.float32),
        grid=(n // IC + ho // RC,),
        in_specs=[
            pl.BlockSpec((IC, h * w, cin),
                         lambda i, _ic=IC: (jnp.minimum(i, n // _ic - 1), 0, 0)),
            pl.BlockSpec((9, cin, cout), lambda i: (0, 0, 0)),
            pl.BlockSpec((1, cout), lambda i: (0, 0)),
            pl.BlockSpec((1, cout), lambda i: (0, 0)),
        ],
        out_specs=pl.BlockSpec((RC, wo, n, cout),
                               lambda i, _ic=IC: (jnp.maximum(i - n // _ic, 0),
                                                  0, 0, 0)),
        scratch_shapes=[
            pltpu.VMEM((n, ho, w, cout), jnp.bfloat16),
            pltpu.VMEM((8, cout), jnp.float32),
        ],
        compiler_params=pltpu.CompilerParams(
            dimension_semantics=("arbitrary",)),
    )(x_flat, w_taps, g_row, b_row)

    return jnp.transpose(out_p, (2, 3, 0, 1))


# speedup vs baseline: 3.2559x; 1.0857x over previous
import jax
import jax.numpy as jnp
from jax.experimental import pallas as pl
from jax.experimental.pallas import tpu as pltpu

EPS = 1e-5


def _make_fused_kernel(n, h, w, ho, wo, L, m_valid, RC, IC):
    def _body(x_ref, w_ref, g_ref, b_ref, o_ref, y_scr, st_scr):
        i = pl.program_id(0)
        cout = o_ref.shape[3]

        @pl.when(i < n // IC)
        def _conv():
            ssum = jnp.zeros((1, cout), jnp.float32)
            sssq = jnp.zeros((1, cout), jnp.float32)
            for k in range(IC):
                xs = x_ref[k]
                cin = xs.shape[1]
                lx = 2 * w + L
                xcat = jnp.concatenate(
                    [xs[kw:kw + lx, :] for kw in range(3)], axis=1)
                acc = jnp.zeros((L, cout), jnp.float32)
                for kh in range(3):
                    off = kh * w
                    acc = acc + jnp.dot(
                        xcat[off:off + L, :], w_ref[kh],
                        preferred_element_type=jnp.float32)

                accp = jnp.concatenate(
                    [acc, jnp.zeros((ho * w - L, cout), jnp.float32)], axis=0)
                y_scr[pl.ds(i * IC + k, 1)] = accp.astype(
                    jnp.bfloat16).reshape(1, ho, w, cout)

                row = jax.lax.broadcasted_iota(jnp.int32, (L, 1), 0)
                mask = (row % w) < wo
                accm = jnp.where(mask, acc, 0.0)
                ssum = ssum + jnp.sum(accm, axis=0, keepdims=True)
                sssq = sssq + jnp.sum(accm * acc, axis=0, keepdims=True)
            sq = jnp.concatenate([ssum, sssq], axis=0)
            prev = jnp.where(i == 0, 0.0, st_scr[0:2])
            st_scr[0:2] = prev + sq

        @pl.when(i >= n // IC)
        def _bn_rows():
            j = i - n // IC
            tot = st_scr[0:1]
            tsq = st_scr[1:2]
            mean = tot / m_valid
            var = jnp.maximum(tsq / m_valid - mean * mean, 0.0)
            inv = jax.lax.rsqrt(var + EPS)
            scale = (g_ref[...] * inv).reshape(1, 1, 1, cout)
            shift = (b_ref[...] - mean * g_ref[...] * inv).reshape(
                1, 1, 1, cout)

            slab = y_scr[:, pl.ds(j * RC, RC), :wo, :]
            z = jnp.maximum(slab.astype(jnp.float32) * scale + shift, 0.0)
            o_ref[...] = jnp.transpose(z, (1, 2, 0, 3))

    return _body


def kernel(x_nchw, w_oihw, bias, gamma, beta):
    del bias
    n, cin, h, w = x_nchw.shape
    cout = w_oihw.shape[0]
    ho, wo = h - 2, w - 2
    L = ho * w - (w - wo)
    RC = next(r for r in (9, 6, 3, 2, 1) if ho % r == 0)
    IC = next(c for c in (4, 2, 1) if n % c == 0)

    x_flat = jnp.transpose(x_nchw, (0, 2, 3, 1)).reshape(n, h * w, cin)

    w_taps = jnp.transpose(w_oihw, (2, 3, 1, 0)).reshape(3, 3 * cin, cout)
    g_row = gamma.reshape(1, cout)
    b_row = beta.reshape(1, cout)

    out_p = pl.pallas_call(
        _make_fused_kernel(n, h, w, ho, wo, L, float(n * ho * wo), RC, IC),
        out_shape=jax.ShapeDtypeStruct((ho, wo, n, cout), jnp.float32),
        grid=(n // IC + ho // RC,),
        in_specs=[
            pl.BlockSpec((IC, h * w, cin),
                         lambda i, _ic=IC: (jnp.minimum(i, n // _ic - 1), 0, 0)),
            pl.BlockSpec((3, 3 * cin, cout), lambda i: (0, 0, 0)),
            pl.BlockSpec((1, cout), lambda i: (0, 0)),
            pl.BlockSpec((1, cout), lambda i: (0, 0)),
        ],
        out_specs=pl.BlockSpec((RC, wo, n, cout),
                               lambda i, _ic=IC: (jnp.maximum(i - n // _ic, 0),
                                                  0, 0, 0)),
        scratch_shapes=[
            pltpu.VMEM((n, ho, w, cout), jnp.bfloat16),
            pltpu.VMEM((8, cout), jnp.float32),
        ],
        compiler_params=pltpu.CompilerParams(
            dimension_semantics=("arbitrary",)),
    )(x_flat, w_taps, g_row, b_row)

    return jnp.transpose(out_p, (2, 3, 0, 1))


# speedup vs baseline: 3.2592x; 1.0010x over previous
import jax
import jax.numpy as jnp
from jax.experimental import pallas as pl
from jax.experimental.pallas import tpu as pltpu

EPS = 1e-5


def _make_fused_kernel(n, h, w, ho, wo, L, m_valid, RC, IC):
    def _body(x_ref, w_ref, g_ref, b_ref, o_ref, y_scr, st_scr):
        i = pl.program_id(0)
        cout = o_ref.shape[3]

        @pl.when(i < n // IC)
        def _conv():
            ssum = jnp.zeros((1, cout), jnp.float32)
            sssq = jnp.zeros((1, cout), jnp.float32)
            for k in range(IC):
                xs = x_ref[k]
                cin = xs.shape[1]
                xcat = jnp.concatenate(
                    [xs[kh * w + kw:kh * w + kw + L, :]
                     for kh in range(3) for kw in range(3)], axis=1)
                acc = jnp.dot(xcat, w_ref[0],
                              preferred_element_type=jnp.float32)

                accp = jnp.concatenate(
                    [acc, jnp.zeros((ho * w - L, cout), jnp.float32)], axis=0)
                y_scr[pl.ds(i * IC + k, 1)] = accp.astype(
                    jnp.bfloat16).reshape(1, ho, w, cout)

                row = jax.lax.broadcasted_iota(jnp.int32, (L, 1), 0)
                mask = (row % w) < wo
                accm = jnp.where(mask, acc, 0.0)
                ssum = ssum + jnp.sum(accm, axis=0, keepdims=True)
                sssq = sssq + jnp.sum(accm * acc, axis=0, keepdims=True)
            sq = jnp.concatenate([ssum, sssq], axis=0)
            prev = jnp.where(i == 0, 0.0, st_scr[0:2])
            st_scr[0:2] = prev + sq

        @pl.when(i >= n // IC)
        def _bn_rows():
            j = i - n // IC
            tot = st_scr[0:1]
            tsq = st_scr[1:2]
            mean = tot / m_valid
            var = jnp.maximum(tsq / m_valid - mean * mean, 0.0)
            inv = jax.lax.rsqrt(var + EPS)
            scale = (g_ref[...] * inv).reshape(1, 1, 1, cout)
            shift = (b_ref[...] - mean * g_ref[...] * inv).reshape(
                1, 1, 1, cout)

            slab = y_scr[:, pl.ds(j * RC, RC), :wo, :]
            z = jnp.maximum(slab.astype(jnp.float32) * scale + shift, 0.0)
            o_ref[...] = jnp.transpose(z, (1, 2, 0, 3))

    return _body


def kernel(x_nchw, w_oihw, bias, gamma, beta):
    del bias
    n, cin, h, w = x_nchw.shape
    cout = w_oihw.shape[0]
    ho, wo = h - 2, w - 2
    L = ho * w - (w - wo)
    RC = next(r for r in (9, 6, 3, 2, 1) if ho % r == 0)
    IC = next(c for c in (4, 2, 1) if n % c == 0)

    x_flat = jnp.transpose(x_nchw, (0, 2, 3, 1)).reshape(n, h * w, cin)

    w_taps = jnp.transpose(w_oihw, (2, 3, 1, 0)).reshape(1, 9 * cin, cout)
    g_row = gamma.reshape(1, cout)
    b_row = beta.reshape(1, cout)

    out_p = pl.pallas_call(
        _make_fused_kernel(n, h, w, ho, wo, L, float(n * ho * wo), RC, IC),
        out_shape=jax.ShapeDtypeStruct((ho, wo, n, cout), jnp.float32),
        grid=(n // IC + ho // RC,),
        in_specs=[
            pl.BlockSpec((IC, h * w, cin),
                         lambda i, _ic=IC: (jnp.minimum(i, n // _ic - 1), 0, 0)),
            pl.BlockSpec((1, 9 * cin, cout), lambda i: (0, 0, 0)),
            pl.BlockSpec((1, cout), lambda i: (0, 0)),
            pl.BlockSpec((1, cout), lambda i: (0, 0)),
        ],
        out_specs=pl.BlockSpec((RC, wo, n, cout),
                               lambda i, _ic=IC: (jnp.maximum(i - n // _ic, 0),
                                                  0, 0, 0)),
        scratch_shapes=[
            pltpu.VMEM((n, ho, w, cout), jnp.bfloat16),
            pltpu.VMEM((8, cout), jnp.float32),
        ],
        compiler_params=pltpu.CompilerParams(
            dimension_semantics=("arbitrary",)),
    )(x_flat, w_taps, g_row, b_row)

    return jnp.transpose(out_p, (2, 3, 0, 1))
